# baseline (speedup 1.0000x reference)
import functools

import jax
import jax.numpy as jnp
from jax import lax
from jax.experimental import pallas as pl
from jax.experimental.pallas import tpu as pltpu
from jax.experimental.pallas import tpu_sc as plsc

_info = plsc.get_sparse_core_info()
_NC, _NS = _info.num_cores, _info.num_subcores
_NW = _NC * _NS


def _make_gather(V, D, B, C=16, NBUF=4):
    assert B % (8 * _NW) == 0
    b_per_w = B // _NW
    assert b_per_w % (C * NBUF) == 0
    n_chunks = b_per_w // C
    mesh = plsc.VectorSubcoreMesh(core_axis_name="c", subcore_axis_name="s")

    @functools.partial(
        pl.kernel,
        mesh=mesh,
        out_type=jax.ShapeDtypeStruct((B, D), jnp.float32),
        scratch_types=[
            pltpu.VMEM((b_per_w,), jnp.int32),
        ]
        + [pltpu.VMEM((C, D), jnp.float32) for _ in range(NBUF)]
        + [pltpu.SemaphoreType.DMA((NBUF,)), pltpu.SemaphoreType.DMA((NBUF,))],
    )
    def gather_kernel(table_hbm, idx_hbm, out_hbm, idx_v, *rest):
        bufs, (gsem, osem) = rest[:NBUF], rest[NBUF:]
        wid = lax.axis_index("s") * _NC + lax.axis_index("c")
        base = wid * b_per_w
        pltpu.sync_copy(idx_hbm.at[pl.ds(base, b_per_w)], idx_v)

        def gather_dma(chunk, j):
            return pltpu.make_async_copy(
                table_hbm.at[idx_v.at[pl.ds(chunk * C, C)]], bufs[j], gsem.at[j]
            )

        def out_dma(chunk, j):
            return pltpu.make_async_copy(
                bufs[j], out_hbm.at[pl.ds(base + chunk * C, C)], osem.at[j]
            )

        K = NBUF - 2

        for c in range(K):
            gather_dma(c, c % NBUF).start()
        for c in range(K):
            gather_dma(c, c % NBUF).wait()
            out_dma(c, c % NBUF).start()
            if c + K - NBUF >= 0:
                out_dma(c + K - NBUF, (c + K) % NBUF).wait()
            gather_dma(c + K, (c + K) % NBUF).start()

        def group_body(g, carry):
            for jp in range(NBUF):
                c = K + g * NBUF + jp
                jb = (K + jp) % NBUF
                gather_dma(c, jb).wait()
                out_dma(c, jb).start()
                jn = (2 * K + jp) % NBUF
                out_dma(c + K - NBUF, jn).wait()
                gather_dma(c + K, jn).start()
            return carry

        assert (n_chunks - 2 * K) % NBUF == 0
        lax.fori_loop(0, (n_chunks - 2 * K) // NBUF, group_body, 0)

        for c in range(n_chunks - K, n_chunks):
            gather_dma(c, c % NBUF).wait()
            out_dma(c, c % NBUF).start()
        for c in range(n_chunks - NBUF, n_chunks):
            out_dma(c, c % NBUF).wait()

    return gather_kernel


def kernel(position_ids, table):
    Bt, S = position_ids.shape
    V, D = table.shape
    idx = position_ids.reshape(Bt * S).astype(jnp.int32)
    out = _make_gather(V, D, Bt * S)(table, idx)
    return out.reshape(Bt, S, D)


# speedup vs baseline: 1.0007x; 1.0007x over previous
import functools

import jax
import jax.numpy as jnp
from jax import lax
from jax.experimental import pallas as pl
from jax.experimental.pallas import tpu as pltpu
from jax.experimental.pallas import tpu_sc as plsc

_info = plsc.get_sparse_core_info()
_NC, _NS = _info.num_cores, _info.num_subcores
_NW = _NC * _NS


def _make_gather(V, D, B, C=16, NBUF=4):
    assert B % (8 * _NW) == 0
    b_per_w = B // _NW
    assert b_per_w % (C * NBUF) == 0
    n_chunks = b_per_w // C
    mesh = plsc.VectorSubcoreMesh(core_axis_name="c", subcore_axis_name="s")

    @functools.partial(
        pl.kernel,
        mesh=mesh,
        out_type=jax.ShapeDtypeStruct((B, D), jnp.float32),
        scratch_types=[
            pltpu.VMEM((b_per_w,), jnp.int32),
        ]
        + [pltpu.VMEM((C, D), jnp.float32) for _ in range(NBUF)]
        + [pltpu.SemaphoreType.DMA((NBUF,)), pltpu.SemaphoreType.DMA((NBUF,))],
    )
    def gather_kernel(table_hbm, idx_hbm, out_hbm, idx_v, *rest):
        bufs, (gsem, osem) = rest[:NBUF], rest[NBUF:]
        wid = lax.axis_index("c") * _NS + lax.axis_index("s")
        base = wid * b_per_w
        pltpu.sync_copy(idx_hbm.at[pl.ds(base, b_per_w)], idx_v)

        def gather_dma(chunk, j):
            return pltpu.make_async_copy(
                table_hbm.at[idx_v.at[pl.ds(chunk * C, C)]], bufs[j], gsem.at[j]
            )

        def out_dma(chunk, j):
            return pltpu.make_async_copy(
                bufs[j], out_hbm.at[pl.ds(base + chunk * C, C)], osem.at[j]
            )

        K = NBUF - 2

        for c in range(K):
            gather_dma(c, c % NBUF).start()
        for c in range(K):
            gather_dma(c, c % NBUF).wait()
            out_dma(c, c % NBUF).start()
            if c + K - NBUF >= 0:
                out_dma(c + K - NBUF, (c + K) % NBUF).wait()
            gather_dma(c + K, (c + K) % NBUF).start()

        def group_body(g, carry):
            for jp in range(NBUF):
                c = K + g * NBUF + jp
                jb = (K + jp) % NBUF
                gather_dma(c, jb).wait()
                out_dma(c, jb).start()
                jn = (2 * K + jp) % NBUF
                out_dma(c + K - NBUF, jn).wait()
                gather_dma(c + K, jn).start()
            return carry

        assert (n_chunks - 2 * K) % NBUF == 0
        lax.fori_loop(0, (n_chunks - 2 * K) // NBUF, group_body, 0)

        for c in range(n_chunks - K, n_chunks):
            gather_dma(c, c % NBUF).wait()
            out_dma(c, c % NBUF).start()
        for c in range(n_chunks - NBUF, n_chunks):
            out_dma(c, c % NBUF).wait()

    return gather_kernel


def kernel(position_ids, table):
    Bt, S = position_ids.shape
    V, D = table.shape
    idx = position_ids.reshape(Bt * S).astype(jnp.int32)
    out = _make_gather(V, D, Bt * S)(table, idx)
    return out.reshape(Bt, S, D)
